# Initial kernel scaffold; baseline (speedup 1.0000x reference)
import jax
import jax.numpy as jnp
from jax.experimental import pallas as pl


def kernel(xyz, xyz_prev, features, features_prev, W1, b1, g1, bt1, W2, b2, g2, bt2):
    raise NotImplementedError("write your pallas kernel here")


# speedup vs baseline: 9.9131x; 9.9131x over previous
import functools

import jax
import jax.numpy as jnp
from jax import lax
from jax.experimental import pallas as pl

_NB = 512


def _knn_mlp1_kernel(xa_ref, ya_ref, feat_ref, fp_ref, w1a_ref, w1b_ref,
                     b1_ref, h1_ref, ssum_ref, ssq_ref):
    b = pl.program_id(0)
    j = pl.program_id(1)
    xa = xa_ref[0]
    ya = ya_ref[0]
    nb = xa.shape[0]
    m = ya.shape[0]
    x2 = jnp.sum(xa[:, 0:3] * xa[:, 0:3], axis=1, keepdims=True)
    d2 = x2 + lax.dot_general(
        xa, ya, (((1,), (1,)), ((), ())),
        preferred_element_type=jnp.float32, precision=lax.Precision.HIGHEST)

    iota = lax.broadcasted_iota(jnp.int32, (nb, m), 1)
    d = d2
    scat = jnp.zeros((nb, m), jnp.float32)
    total = jnp.zeros((nb, 1), jnp.float32)
    for _ in range(3):
        mn = jnp.min(d, axis=1, keepdims=True)
        am = jnp.min(jnp.where(d == mn, iota, m), axis=1,
                     keepdims=True)
        dist = jnp.sqrt(jnp.maximum(mn, 1e-12))
        inv = 1.0 / (dist + 1e-8)
        sel = iota == am
        scat = scat + jnp.where(sel, inv, 0.0)
        total = total + inv
        d = jnp.where(sel, jnp.float32(jnp.inf), d)
    scat = scat * (1.0 / total)

    interp = lax.dot_general(
        scat, fp_ref[0], (((1,), (1,)), ((), ())),
        preferred_element_type=jnp.float32, precision=lax.Precision.HIGHEST)
    h1 = (lax.dot_general(w1a_ref[...], interp, (((1,), (1,)), ((), ())),
                          preferred_element_type=jnp.float32,
                          precision=lax.Precision.HIGHEST)
          + lax.dot_general(w1b_ref[...], feat_ref[0],
                            (((1,), (0,)), ((), ())),
                            preferred_element_type=jnp.float32,
                            precision=lax.Precision.HIGHEST)
          + b1_ref[...])
    h1_ref[0] = h1

    @pl.when((b == 0) & (j == 0))
    def _():
        ssum_ref[...] = jnp.zeros_like(ssum_ref)
        ssq_ref[...] = jnp.zeros_like(ssq_ref)

    ssum_ref[...] += jnp.sum(h1, axis=1, keepdims=True)
    ssq_ref[...] += jnp.sum(h1 * h1, axis=1, keepdims=True)


def _mlp2_kernel(cnt_inv, h1_ref, s1_ref, q1_ref, g1_ref, bt1_ref, w2_ref,
                 b2_ref, h2_ref, ssum_ref, ssq_ref):
    b = pl.program_id(0)
    j = pl.program_id(1)
    mean = s1_ref[...] * cnt_inv
    var = q1_ref[...] * cnt_inv - mean * mean
    scale = g1_ref[...] * lax.rsqrt(var + 1e-5)
    shift = bt1_ref[...] - mean * scale
    r = jnp.maximum(h1_ref[0] * scale + shift, 0.0)
    h2 = lax.dot_general(w2_ref[...], r, (((1,), (0,)), ((), ())),
                         preferred_element_type=jnp.float32,
                         precision=lax.Precision.HIGHEST) + b2_ref[...]
    h2_ref[0] = h2

    @pl.when((b == 0) & (j == 0))
    def _():
        ssum_ref[...] = jnp.zeros_like(ssum_ref)
        ssq_ref[...] = jnp.zeros_like(ssq_ref)

    ssum_ref[...] += jnp.sum(h2, axis=1, keepdims=True)
    ssq_ref[...] += jnp.sum(h2 * h2, axis=1, keepdims=True)


def _bn2_kernel(cnt_inv, h2_ref, s2_ref, q2_ref, g2_ref, bt2_ref, out_ref):
    mean = s2_ref[...] * cnt_inv
    var = q2_ref[...] * cnt_inv - mean * mean
    scale = g2_ref[...] * lax.rsqrt(var + 1e-5)
    shift = bt2_ref[...] - mean * scale
    out_ref[0] = jnp.maximum(h2_ref[0] * scale + shift, 0.0)


def kernel(xyz, xyz_prev, features, features_prev, W1, b1, g1, bt1, W2, b2,
           g2, bt2):
    B, N, _ = xyz.shape
    M = xyz_prev.shape[1]
    C = features.shape[1]
    Cp = features_prev.shape[1]
    d1 = W1.shape[0]
    d2o = W2.shape[0]
    nb = _NB
    nblk = N // nb

    xa = jnp.concatenate(
        [xyz, jnp.ones((B, N, 1), jnp.float32)], axis=2)
    ya = jnp.concatenate(
        [-2.0 * xyz_prev,
         jnp.sum(xyz_prev * xyz_prev, axis=2, keepdims=True)], axis=2)
    w1a = W1[:, :Cp]
    w1b = W1[:, Cp:]
    b1c = b1.reshape(d1, 1)
    g1c = g1.reshape(d1, 1)
    bt1c = bt1.reshape(d1, 1)
    b2c = b2.reshape(d2o, 1)
    g2c = g2.reshape(d2o, 1)
    bt2c = bt2.reshape(d2o, 1)
    cnt_inv = 1.0 / float(B * N)

    grid = (B, nblk)
    h1, s1, q1 = pl.pallas_call(
        _knn_mlp1_kernel,
        grid=grid,
        in_specs=[
            pl.BlockSpec((1, nb, 4), lambda b, j: (b, j, 0)),
            pl.BlockSpec((1, M, 4), lambda b, j: (b, 0, 0)),
            pl.BlockSpec((1, C, nb), lambda b, j: (b, 0, j)),
            pl.BlockSpec((1, Cp, M), lambda b, j: (b, 0, 0)),
            pl.BlockSpec((d1, Cp), lambda b, j: (0, 0)),
            pl.BlockSpec((d1, C), lambda b, j: (0, 0)),
            pl.BlockSpec((d1, 1), lambda b, j: (0, 0)),
        ],
        out_specs=[
            pl.BlockSpec((1, d1, nb), lambda b, j: (b, 0, j)),
            pl.BlockSpec((d1, 1), lambda b, j: (0, 0)),
            pl.BlockSpec((d1, 1), lambda b, j: (0, 0)),
        ],
        out_shape=[
            jax.ShapeDtypeStruct((B, d1, N), jnp.float32),
            jax.ShapeDtypeStruct((d1, 1), jnp.float32),
            jax.ShapeDtypeStruct((d1, 1), jnp.float32),
        ],
    )(xa, ya, features, features_prev, w1a, w1b, b1c)

    h2, s2, q2 = pl.pallas_call(
        functools.partial(_mlp2_kernel, cnt_inv),
        grid=grid,
        in_specs=[
            pl.BlockSpec((1, d1, nb), lambda b, j: (b, 0, j)),
            pl.BlockSpec((d1, 1), lambda b, j: (0, 0)),
            pl.BlockSpec((d1, 1), lambda b, j: (0, 0)),
            pl.BlockSpec((d1, 1), lambda b, j: (0, 0)),
            pl.BlockSpec((d1, 1), lambda b, j: (0, 0)),
            pl.BlockSpec((d2o, d1), lambda b, j: (0, 0)),
            pl.BlockSpec((d2o, 1), lambda b, j: (0, 0)),
        ],
        out_specs=[
            pl.BlockSpec((1, d2o, nb), lambda b, j: (b, 0, j)),
            pl.BlockSpec((d2o, 1), lambda b, j: (0, 0)),
            pl.BlockSpec((d2o, 1), lambda b, j: (0, 0)),
        ],
        out_shape=[
            jax.ShapeDtypeStruct((B, d2o, N), jnp.float32),
            jax.ShapeDtypeStruct((d2o, 1), jnp.float32),
            jax.ShapeDtypeStruct((d2o, 1), jnp.float32),
        ],
    )(h1, s1, q1, g1c, bt1c, W2, b2c)

    out = pl.pallas_call(
        functools.partial(_bn2_kernel, cnt_inv),
        grid=grid,
        in_specs=[
            pl.BlockSpec((1, d2o, nb), lambda b, j: (b, 0, j)),
            pl.BlockSpec((d2o, 1), lambda b, j: (0, 0)),
            pl.BlockSpec((d2o, 1), lambda b, j: (0, 0)),
            pl.BlockSpec((d2o, 1), lambda b, j: (0, 0)),
            pl.BlockSpec((d2o, 1), lambda b, j: (0, 0)),
        ],
        out_specs=pl.BlockSpec((1, d2o, nb), lambda b, j: (b, 0, j)),
        out_shape=jax.ShapeDtypeStruct((B, d2o, N), jnp.float32),
    )(h2, s2, q2, g2c, bt2c)
    return out


# speedup vs baseline: 16.1413x; 1.6283x over previous
import functools

import jax
import jax.numpy as jnp
from jax import lax
from jax.experimental import pallas as pl

_NB = 512


def _knn_mlp1_kernel(xa_ref, ya_ref, feat_ref, fp_ref, w1a_ref, w1b_ref,
                     b1_ref, h1_ref, ssum_ref, ssq_ref):
    b = pl.program_id(0)
    j = pl.program_id(1)
    xa = xa_ref[0]
    ya = ya_ref[0]
    nb = xa.shape[0]
    m = ya.shape[0]
    x2 = jnp.sum(xa[:, 0:3] * xa[:, 0:3], axis=1, keepdims=True)
    d2 = x2 + lax.dot_general(
        xa, ya, (((1,), (1,)), ((), ())),
        preferred_element_type=jnp.float32, precision=lax.Precision.HIGHEST)

    iota = lax.broadcasted_iota(jnp.int32, (nb, m), 1)
    d = d2
    scat = jnp.zeros((nb, m), jnp.float32)
    total = jnp.zeros((nb, 1), jnp.float32)
    for _ in range(3):
        mn = jnp.min(d, axis=1, keepdims=True)
        am = jnp.min(jnp.where(d == mn, iota, m), axis=1,
                     keepdims=True)
        dist = jnp.sqrt(jnp.maximum(mn, 1e-12))
        inv = 1.0 / (dist + 1e-8)
        sel = iota == am
        scat = scat + jnp.where(sel, inv, 0.0)
        total = total + inv
        d = jnp.where(sel, jnp.float32(jnp.inf), d)
    scat = scat * (1.0 / total)

    interp = lax.dot_general(
        scat, fp_ref[0], (((1,), (1,)), ((), ())),
        preferred_element_type=jnp.float32, precision=lax.Precision.DEFAULT)
    h1 = (lax.dot_general(w1a_ref[...], interp, (((1,), (1,)), ((), ())),
                          preferred_element_type=jnp.float32,
                          precision=lax.Precision.DEFAULT)
          + lax.dot_general(w1b_ref[...], feat_ref[0],
                            (((1,), (0,)), ((), ())),
                            preferred_element_type=jnp.float32,
                            precision=lax.Precision.DEFAULT)
          + b1_ref[...])
    h1_ref[0] = h1

    @pl.when((b == 0) & (j == 0))
    def _():
        ssum_ref[...] = jnp.zeros_like(ssum_ref)
        ssq_ref[...] = jnp.zeros_like(ssq_ref)

    ssum_ref[...] += jnp.sum(h1, axis=1, keepdims=True)
    ssq_ref[...] += jnp.sum(h1 * h1, axis=1, keepdims=True)


def _mlp2_kernel(cnt_inv, h1_ref, s1_ref, q1_ref, g1_ref, bt1_ref, w2_ref,
                 b2_ref, h2_ref, ssum_ref, ssq_ref):
    b = pl.program_id(0)
    j = pl.program_id(1)
    mean = s1_ref[...] * cnt_inv
    var = q1_ref[...] * cnt_inv - mean * mean
    scale = g1_ref[...] * lax.rsqrt(var + 1e-5)
    shift = bt1_ref[...] - mean * scale
    r = jnp.maximum(h1_ref[0] * scale + shift, 0.0)
    h2 = lax.dot_general(w2_ref[...], r, (((1,), (0,)), ((), ())),
                         preferred_element_type=jnp.float32,
                         precision=lax.Precision.DEFAULT) + b2_ref[...]
    h2_ref[0] = h2

    @pl.when((b == 0) & (j == 0))
    def _():
        ssum_ref[...] = jnp.zeros_like(ssum_ref)
        ssq_ref[...] = jnp.zeros_like(ssq_ref)

    ssum_ref[...] += jnp.sum(h2, axis=1, keepdims=True)
    ssq_ref[...] += jnp.sum(h2 * h2, axis=1, keepdims=True)


def _bn2_kernel(cnt_inv, h2_ref, s2_ref, q2_ref, g2_ref, bt2_ref, out_ref):
    mean = s2_ref[...] * cnt_inv
    var = q2_ref[...] * cnt_inv - mean * mean
    scale = g2_ref[...] * lax.rsqrt(var + 1e-5)
    shift = bt2_ref[...] - mean * scale
    out_ref[0] = jnp.maximum(h2_ref[0] * scale + shift, 0.0)


def kernel(xyz, xyz_prev, features, features_prev, W1, b1, g1, bt1, W2, b2,
           g2, bt2):
    B, N, _ = xyz.shape
    M = xyz_prev.shape[1]
    C = features.shape[1]
    Cp = features_prev.shape[1]
    d1 = W1.shape[0]
    d2o = W2.shape[0]
    nb = _NB
    nblk = N // nb

    xa = jnp.concatenate(
        [xyz, jnp.ones((B, N, 1), jnp.float32)], axis=2)
    ya = jnp.concatenate(
        [-2.0 * xyz_prev,
         jnp.sum(xyz_prev * xyz_prev, axis=2, keepdims=True)], axis=2)
    w1a = W1[:, :Cp]
    w1b = W1[:, Cp:]
    b1c = b1.reshape(d1, 1)
    g1c = g1.reshape(d1, 1)
    bt1c = bt1.reshape(d1, 1)
    b2c = b2.reshape(d2o, 1)
    g2c = g2.reshape(d2o, 1)
    bt2c = bt2.reshape(d2o, 1)
    cnt_inv = 1.0 / float(B * N)

    grid = (B, nblk)
    h1, s1, q1 = pl.pallas_call(
        _knn_mlp1_kernel,
        grid=grid,
        in_specs=[
            pl.BlockSpec((1, nb, 4), lambda b, j: (b, j, 0)),
            pl.BlockSpec((1, M, 4), lambda b, j: (b, 0, 0)),
            pl.BlockSpec((1, C, nb), lambda b, j: (b, 0, j)),
            pl.BlockSpec((1, Cp, M), lambda b, j: (b, 0, 0)),
            pl.BlockSpec((d1, Cp), lambda b, j: (0, 0)),
            pl.BlockSpec((d1, C), lambda b, j: (0, 0)),
            pl.BlockSpec((d1, 1), lambda b, j: (0, 0)),
        ],
        out_specs=[
            pl.BlockSpec((1, d1, nb), lambda b, j: (b, 0, j)),
            pl.BlockSpec((d1, 1), lambda b, j: (0, 0)),
            pl.BlockSpec((d1, 1), lambda b, j: (0, 0)),
        ],
        out_shape=[
            jax.ShapeDtypeStruct((B, d1, N), jnp.float32),
            jax.ShapeDtypeStruct((d1, 1), jnp.float32),
            jax.ShapeDtypeStruct((d1, 1), jnp.float32),
        ],
    )(xa, ya, features, features_prev, w1a, w1b, b1c)

    h2, s2, q2 = pl.pallas_call(
        functools.partial(_mlp2_kernel, cnt_inv),
        grid=grid,
        in_specs=[
            pl.BlockSpec((1, d1, nb), lambda b, j: (b, 0, j)),
            pl.BlockSpec((d1, 1), lambda b, j: (0, 0)),
            pl.BlockSpec((d1, 1), lambda b, j: (0, 0)),
            pl.BlockSpec((d1, 1), lambda b, j: (0, 0)),
            pl.BlockSpec((d1, 1), lambda b, j: (0, 0)),
            pl.BlockSpec((d2o, d1), lambda b, j: (0, 0)),
            pl.BlockSpec((d2o, 1), lambda b, j: (0, 0)),
        ],
        out_specs=[
            pl.BlockSpec((1, d2o, nb), lambda b, j: (b, 0, j)),
            pl.BlockSpec((d2o, 1), lambda b, j: (0, 0)),
            pl.BlockSpec((d2o, 1), lambda b, j: (0, 0)),
        ],
        out_shape=[
            jax.ShapeDtypeStruct((B, d2o, N), jnp.float32),
            jax.ShapeDtypeStruct((d2o, 1), jnp.float32),
            jax.ShapeDtypeStruct((d2o, 1), jnp.float32),
        ],
    )(h1, s1, q1, g1c, bt1c, W2, b2c)

    out = pl.pallas_call(
        functools.partial(_bn2_kernel, cnt_inv),
        grid=grid,
        in_specs=[
            pl.BlockSpec((1, d2o, nb), lambda b, j: (b, 0, j)),
            pl.BlockSpec((d2o, 1), lambda b, j: (0, 0)),
            pl.BlockSpec((d2o, 1), lambda b, j: (0, 0)),
            pl.BlockSpec((d2o, 1), lambda b, j: (0, 0)),
            pl.BlockSpec((d2o, 1), lambda b, j: (0, 0)),
        ],
        out_specs=pl.BlockSpec((1, d2o, nb), lambda b, j: (b, 0, j)),
        out_shape=jax.ShapeDtypeStruct((B, d2o, N), jnp.float32),
    )(h2, s2, q2, g2c, bt2c)
    return out


# speedup vs baseline: 25.7739x; 1.5968x over previous
import functools

import jax
import jax.numpy as jnp
from jax import lax
from jax.experimental import pallas as pl
from jax.experimental.pallas import tpu as pltpu

_NB = 1024


def _fused_kernel(cnt_inv, nblk, xa_ref, ya_ref, feat_ref, fp_ref, w1a_ref,
                  w1b_ref, b1_ref, g1_ref, bt1_ref, w2_ref, b2_ref, g2_ref,
                  bt2_ref, out_ref, h_ref, s1_ref, q1_ref, s2_ref, q2_ref):
    p = pl.program_id(0)
    b = pl.program_id(1)
    j = pl.program_id(2)
    blk = b * nblk + j

    @pl.when(p == 0)
    def _phase0():
        xa = xa_ref[0]
        ya = ya_ref[0]
        nb = xa.shape[0]
        m = ya.shape[0]
        x2 = jnp.sum(xa[:, 0:3] * xa[:, 0:3], axis=1, keepdims=True)
        d = lax.dot_general(
            xa, ya, (((1,), (1,)), ((), ())),
            preferred_element_type=jnp.float32,
            precision=lax.Precision.HIGHEST)

        scat = jnp.zeros((nb, m), jnp.float32)
        total = jnp.zeros((nb, 1), jnp.float32)
        for k in range(3):
            mn = jnp.min(d, axis=1, keepdims=True)
            inv = lax.rsqrt(jnp.maximum(x2 + mn, 1e-12))
            sel = d == mn
            scat = jnp.where(sel, inv, scat)
            total = total + inv
            if k < 2:
                d = jnp.where(sel, jnp.float32(jnp.inf), d)

        interp = lax.dot_general(
            scat, fp_ref[0], (((1,), (1,)), ((), ())),
            preferred_element_type=jnp.float32,
            precision=lax.Precision.DEFAULT) * (1.0 / total)
        h1 = (lax.dot_general(w1a_ref[...], interp,
                              (((1,), (1,)), ((), ())),
                              preferred_element_type=jnp.float32,
                              precision=lax.Precision.DEFAULT)
              + lax.dot_general(w1b_ref[...], feat_ref[0],
                                (((1,), (0,)), ((), ())),
                                preferred_element_type=jnp.float32,
                                precision=lax.Precision.DEFAULT)
              + b1_ref[...])
        h_ref[blk] = h1

        @pl.when((b == 0) & (j == 0))
        def _():
            s1_ref[...] = jnp.zeros_like(s1_ref)
            q1_ref[...] = jnp.zeros_like(q1_ref)

        s1_ref[...] += jnp.sum(h1, axis=1, keepdims=True)
        q1_ref[...] += jnp.sum(h1 * h1, axis=1, keepdims=True)

    @pl.when(p == 1)
    def _phase1():
        mean = s1_ref[...] * cnt_inv
        var = q1_ref[...] * cnt_inv - mean * mean
        scale = g1_ref[...] * lax.rsqrt(var + 1e-5)
        shift = bt1_ref[...] - mean * scale
        r = jnp.maximum(h_ref[blk] * scale + shift, 0.0)
        h2 = lax.dot_general(w2_ref[...], r, (((1,), (0,)), ((), ())),
                             preferred_element_type=jnp.float32,
                             precision=lax.Precision.DEFAULT) + b2_ref[...]
        h_ref[blk] = h2

        @pl.when((b == 0) & (j == 0))
        def _():
            s2_ref[...] = jnp.zeros_like(s2_ref)
            q2_ref[...] = jnp.zeros_like(q2_ref)

        s2_ref[...] += jnp.sum(h2, axis=1, keepdims=True)
        q2_ref[...] += jnp.sum(h2 * h2, axis=1, keepdims=True)

    @pl.when(p == 2)
    def _phase2():
        mean = s2_ref[...] * cnt_inv
        var = q2_ref[...] * cnt_inv - mean * mean
        scale = g2_ref[...] * lax.rsqrt(var + 1e-5)
        shift = bt2_ref[...] - mean * scale
        out_ref[0] = jnp.maximum(h_ref[blk] * scale + shift, 0.0)


def kernel(xyz, xyz_prev, features, features_prev, W1, b1, g1, bt1, W2, b2,
           g2, bt2):
    B, N, _ = xyz.shape
    M = xyz_prev.shape[1]
    C = features.shape[1]
    Cp = features_prev.shape[1]
    d1 = W1.shape[0]
    d2o = W2.shape[0]
    nb = _NB
    nblk = N // nb

    xa = jnp.concatenate(
        [xyz, jnp.ones((B, N, 1), jnp.float32)], axis=2)
    ya = jnp.concatenate(
        [-2.0 * xyz_prev,
         jnp.sum(xyz_prev * xyz_prev, axis=2, keepdims=True)], axis=2)
    w1a = W1[:, :Cp]
    w1b = W1[:, Cp:]
    b1c = b1.reshape(d1, 1)
    g1c = g1.reshape(d1, 1)
    bt1c = bt1.reshape(d1, 1)
    b2c = b2.reshape(d2o, 1)
    g2c = g2.reshape(d2o, 1)
    bt2c = bt2.reshape(d2o, 1)
    cnt_inv = 1.0 / float(B * N)

    zero = lambda p, b, j: (0, 0)
    out = pl.pallas_call(
        functools.partial(_fused_kernel, cnt_inv, nblk),
        grid=(3, B, nblk),
        in_specs=[
            pl.BlockSpec((1, nb, 4),
                         lambda p, b, j: (jnp.where(p == 0, b, 0),
                                          jnp.where(p == 0, j, 0), 0)),
            pl.BlockSpec((1, M, 4),
                         lambda p, b, j: (jnp.where(p == 0, b, 0), 0, 0)),
            pl.BlockSpec((1, C, nb),
                         lambda p, b, j: (jnp.where(p == 0, b, 0), 0,
                                          jnp.where(p == 0, j, 0))),
            pl.BlockSpec((1, Cp, M),
                         lambda p, b, j: (jnp.where(p == 0, b, 0), 0, 0)),
            pl.BlockSpec((d1, Cp), zero),
            pl.BlockSpec((d1, C), zero),
            pl.BlockSpec((d1, 1), zero),
            pl.BlockSpec((d1, 1), zero),
            pl.BlockSpec((d1, 1), zero),
            pl.BlockSpec((d2o, d1), zero),
            pl.BlockSpec((d2o, 1), zero),
            pl.BlockSpec((d2o, 1), zero),
            pl.BlockSpec((d2o, 1), zero),
        ],
        out_specs=pl.BlockSpec(
            (1, d2o, nb),
            lambda p, b, j: (jnp.where(p == 2, b, 0), 0,
                             jnp.where(p == 2, j, 0))),
        out_shape=jax.ShapeDtypeStruct((B, d2o, N), jnp.float32),
        scratch_shapes=[
            pltpu.VMEM((B * nblk, d1, nb), jnp.float32),
            pltpu.VMEM((d1, 1), jnp.float32),
            pltpu.VMEM((d1, 1), jnp.float32),
            pltpu.VMEM((d2o, 1), jnp.float32),
            pltpu.VMEM((d2o, 1), jnp.float32),
        ],
    )(xa, ya, features, features_prev, w1a, w1b, b1c, g1c, bt1c, W2, b2c,
      g2c, bt2c)
    return out
